# Initial kernel scaffold; baseline (speedup 1.0000x reference)
import functools

import jax
import jax.numpy as jnp
from jax import lax
from jax.experimental import pallas as pl
from jax.experimental.pallas import tpu as pltpu
from jax.experimental.pallas import tpu_sc as plsc

N = 10000
E = 320000
H = 128
NUM_BESSEL = 8
R_MAX = 5.0
NUM_GRAPHS = 32
AVG_NEIGH = 32.0
SQRT3 = 3.0 ** 0.5

NC = 2
NS = 16
NW = NC * NS
PER_W = E // NW
CHUNK = 80
NCHUNK = PER_W // CHUNK
ROWS_PER_SUB = N // NS

_SC_MESH = plsc.VectorSubcoreMesh(core_axis_name="c", subcore_axis_name="s")


def _scg_body(pos_hbm, snd_hbm, rcv_hbm, out_s, out_r,
              sidx, ridx, bs, br, sem_a, sem_b):
  cid = lax.axis_index("c")
  sid = lax.axis_index("s")
  wid = cid * NS + sid

  def chunk(c, _):
    base = wid * PER_W + c * CHUNK
    pltpu.sync_copy(snd_hbm.at[pl.ds(base, CHUNK)], sidx)
    pltpu.sync_copy(rcv_hbm.at[pl.ds(base, CHUNK)], ridx)
    cp_a = pltpu.async_copy(pos_hbm.at[sidx], bs, sem_a)
    cp_b = pltpu.async_copy(pos_hbm.at[ridx], br, sem_b)
    cp_a.wait()
    cp_b.wait()
    pltpu.sync_copy(bs, out_s.at[pl.ds(base, CHUNK)])
    pltpu.sync_copy(br, out_r.at[pl.ds(base, CHUNK)])
    return 0

  lax.fori_loop(0, NCHUNK, chunk, 0)


_scg = pl.kernel(
    _scg_body,
    out_type=(
        jax.ShapeDtypeStruct((E, 16), jnp.float32),
        jax.ShapeDtypeStruct((E, 16), jnp.float32),
    ),
    mesh=_SC_MESH,
    scratch_types=[
        pltpu.VMEM((CHUNK,), jnp.int32),
        pltpu.VMEM((CHUNK,), jnp.int32),
        pltpu.VMEM((CHUNK, 16), jnp.float32),
        pltpu.VMEM((CHUNK, 16), jnp.float32),
        pltpu.SemaphoreType.DMA,
        pltpu.SemaphoreType.DMA,
    ],
)


def _scatter_body(h_hbm, rp_hbm, snd_hbm, rcv_hbm, zeros_hbm, out_hbm,
                  sidx, ridx, hbuf, rbuf, sem_g, acc):
  cid = lax.axis_index("c")
  sid = lax.axis_index("s")
  wid = cid * NS + sid

  row0 = sid * ROWS_PER_SUB
  pltpu.sync_copy(zeros_hbm.at[pl.ds(row0, ROWS_PER_SUB)],
                  acc.at[pl.ds(row0, ROWS_PER_SUB)])
  plsc.subcore_barrier()

  def chunk(c, _):
    base = wid * PER_W + c * CHUNK
    pltpu.sync_copy(snd_hbm.at[pl.ds(base, CHUNK)], sidx)
    pltpu.sync_copy(rcv_hbm.at[pl.ds(base, CHUNK)], ridx)
    pltpu.async_copy(h_hbm.at[sidx], hbuf, sem_g).wait()
    pltpu.sync_copy(rp_hbm.at[pl.ds(base, CHUNK)], rbuf)

    def mul_row(i, _):
      for k in range(H // 16):
        sl = pl.ds(k * 16, 16)
        hbuf[i, sl] = hbuf[i, sl] * rbuf[i, sl]
      return 0

    lax.fori_loop(0, CHUNK, mul_row, 0)
    pltpu.sync_copy(hbuf, acc.at[ridx], add=True)
    return 0

  lax.fori_loop(0, NCHUNK, chunk, 0)
  plsc.subcore_barrier()
  pltpu.sync_copy(acc.at[pl.ds(row0, ROWS_PER_SUB)],
                  out_hbm.at[cid, pl.ds(row0, ROWS_PER_SUB)])


_scatter = pl.kernel(
    _scatter_body,
    out_type=jax.ShapeDtypeStruct((NC, N, H), jnp.float32),
    mesh=_SC_MESH,
    scratch_types=[
        pltpu.VMEM((CHUNK,), jnp.int32),
        pltpu.VMEM((CHUNK,), jnp.int32),
        pltpu.VMEM((CHUNK, H), jnp.float32),
        pltpu.VMEM((CHUNK, H), jnp.float32),
        pltpu.SemaphoreType.DMA,
        pltpu.VMEM_SHARED((N, H), jnp.float32),
    ],
)


EBLK = 2000


def _edge_tc_body(ps_ref, pr_ref, sh_ref, w1a_ref, w2a_ref, w1b_ref, w2b_ref,
                  wv_ref, w0_ref, rp0_ref, rp1_ref):
  d = pr_ref[...] - ps_ref[...] + sh_ref[...]
  len2 = jnp.sum(d * d, axis=1, keepdims=True)
  ln = jnp.sqrt(len2)
  lc = jnp.maximum(ln, 1e-9)
  x = ln * (1.0 / R_MAX)
  env = 1.0 + x ** 5 * (-21.0 + x * (35.0 - 15.0 * x))
  cut = jnp.where(x < 1.0, env, 0.0)
  nn = (lax.broadcasted_iota(jnp.float32, (1, NUM_BESSEL), 1) + 1.0) \
      * (jnp.pi / R_MAX)
  pref = (2.0 / R_MAX) ** 0.5
  ef = jnp.sin(lc * nn) * ((pref * cut) / lc)

  inv_lc = 1.0 / lc

  def radial(w1, w2, row):
    u = jnp.dot(ef, w1, preferred_element_type=jnp.float32)
    a = u * jax.nn.sigmoid(u)
    r = jnp.dot(a, w2, preferred_element_type=jnp.float32)
    dotd = jnp.sum(d * wv_ref[row:row + 1, :], axis=1, keepdims=True)
    sh = w0_ref[row] + dotd * inv_lc
    return r * (sh * (1.0 / AVG_NEIGH))

  rp0_ref[...] = radial(w1a_ref[...], w2a_ref[...], 0)
  rp1_ref[...] = radial(w1b_ref[...], w2b_ref[...], 1)


def _edge_tc(ps, pr, shp, w1a, w2a, w1b, w2b, wv, w0):
  g = E // EBLK
  eb = lambda i: (i, 0)
  wfull = lambda i: (0, 0)
  return pl.pallas_call(
      _edge_tc_body,
      grid=(g,),
      in_specs=[
          pl.BlockSpec((EBLK, 16), eb),
          pl.BlockSpec((EBLK, 16), eb),
          pl.BlockSpec((EBLK, 16), eb),
          pl.BlockSpec((NUM_BESSEL, 64), wfull),
          pl.BlockSpec((64, H), wfull),
          pl.BlockSpec((NUM_BESSEL, 64), wfull),
          pl.BlockSpec((64, H), wfull),
          pl.BlockSpec((2, 16), wfull),
          pl.BlockSpec(memory_space=pltpu.SMEM),
      ],
      out_specs=[
          pl.BlockSpec((EBLK, H), eb),
          pl.BlockSpec((EBLK, H), eb),
      ],
      out_shape=[
          jax.ShapeDtypeStruct((E, H), jnp.float32),
          jax.ShapeDtypeStruct((E, H), jnp.float32),
      ],
  )(ps, pr, shp, w1a, w2a, w1b, w2b, wv, w0)


NBLK = 1000


def _h0_body(na_ref, w_ref, out_ref):
  out_ref[...] = jnp.dot(na_ref[...], w_ref[...],
                         preferred_element_type=jnp.float32)


def _h0(na_p, wupe):
  return pl.pallas_call(
      _h0_body,
      grid=(N // NBLK,),
      in_specs=[
          pl.BlockSpec((NBLK, 16), lambda i: (i, 0)),
          pl.BlockSpec((16, H), lambda i: (0, 0)),
      ],
      out_specs=pl.BlockSpec((NBLK, H), lambda i: (i, 0)),
      out_shape=jax.ShapeDtypeStruct((N, H), jnp.float32),
  )(na_p, wupe)


def _c0_body(pagg_ref, na_ref, batch_ref, wout_ref, wsce_ref, wr0_ref,
             ae_ref, wup1_ref, nf1_ref, h1_ref, e_ref):
  agg = pagg_ref[0] + pagg_ref[1]
  nf1 = jnp.dot(agg, wout_ref[...], preferred_element_type=jnp.float32) \
      + jnp.dot(na_ref[...], wsce_ref[...], preferred_element_type=jnp.float32)
  nf1_ref[...] = nf1
  h1_ref[...] = jnp.dot(nf1, wup1_ref[...], preferred_element_type=jnp.float32)
  en = jnp.sum(nf1 * wr0_ref[...], axis=1) \
      + jnp.sum(na_ref[...] * ae_ref[...], axis=1)
  gid = lax.broadcasted_iota(jnp.int32, (NBLK, H), 1)
  mask = batch_ref[...] == gid
  contrib = jnp.sum(jnp.where(mask, en[:, None], 0.0), axis=0)

  @pl.when(pl.program_id(0) == 0)
  def _():
    e_ref[...] = jnp.zeros_like(e_ref)

  e_ref[...] += contrib[None, :]


def _c0(pagg, na_p, batch2d, wout0, wsce, wr0row, aerow, wup1):
  nb = lambda i: (i, 0)
  wfull = lambda i: (0, 0)
  return pl.pallas_call(
      _c0_body,
      grid=(N // NBLK,),
      in_specs=[
          pl.BlockSpec((NC, NBLK, H), lambda i: (0, i, 0)),
          pl.BlockSpec((NBLK, 16), nb),
          pl.BlockSpec((NBLK, 1), nb),
          pl.BlockSpec((H, H), wfull),
          pl.BlockSpec((16, H), wfull),
          pl.BlockSpec((1, H), wfull),
          pl.BlockSpec((1, 16), wfull),
          pl.BlockSpec((H, H), wfull),
      ],
      out_specs=[
          pl.BlockSpec((NBLK, H), nb),
          pl.BlockSpec((NBLK, H), nb),
          pl.BlockSpec((1, H), wfull),
      ],
      out_shape=[
          jax.ShapeDtypeStruct((N, H), jnp.float32),
          jax.ShapeDtypeStruct((N, H), jnp.float32),
          jax.ShapeDtypeStruct((1, H), jnp.float32),
      ],
  )(pagg, na_p, batch2d, wout0, wsce, wr0row, aerow, wup1)


def _c1_body(pagg_ref, nf1_ref, batch_ref, wout_ref, wsc_ref, wr1a_ref,
             wr1b_ref, e_ref):
  agg = pagg_ref[0] + pagg_ref[1]
  nf2 = jnp.dot(agg, wout_ref[...], preferred_element_type=jnp.float32) \
      + jnp.dot(nf1_ref[...], wsc_ref[...], preferred_element_type=jnp.float32)
  t = jnp.dot(nf2, wr1a_ref[...], preferred_element_type=jnp.float32)
  t = t * jax.nn.sigmoid(t)
  en = jnp.sum(t * wr1b_ref[...], axis=1)
  gid = lax.broadcasted_iota(jnp.int32, (NBLK, H), 1)
  mask = batch_ref[...] == gid
  contrib = jnp.sum(jnp.where(mask, en[:, None], 0.0), axis=0)

  @pl.when(pl.program_id(0) == 0)
  def _():
    e_ref[...] = jnp.zeros_like(e_ref)

  e_ref[...] += contrib[None, :]


def _c1(pagg, nf1, batch2d, wout1, wsc1, wr1a, wr1brow):
  nb = lambda i: (i, 0)
  wfull = lambda i: (0, 0)
  return pl.pallas_call(
      _c1_body,
      grid=(N // NBLK,),
      in_specs=[
          pl.BlockSpec((NC, NBLK, H), lambda i: (0, i, 0)),
          pl.BlockSpec((NBLK, H), nb),
          pl.BlockSpec((NBLK, 1), nb),
          pl.BlockSpec((H, H), wfull),
          pl.BlockSpec((H, H), wfull),
          pl.BlockSpec((H, 16), wfull),
          pl.BlockSpec((1, 16), wfull),
      ],
      out_specs=pl.BlockSpec((1, H), wfull),
      out_shape=jax.ShapeDtypeStruct((1, H), jnp.float32),
  )(pagg, nf1, batch2d, wout1, wsc1, wr1a, wr1brow)


def kernel(positions, node_attrs, shifts, atomic_energies, W_embed, W_up,
           R_W1, R_W2, w_sh, W_out, W_sc, W_r0, W_r1a, W_r1b,
           edge_index, batch):
  f32 = jnp.float32
  snd = edge_index[0].astype(jnp.int32)
  rcv = edge_index[1].astype(jnp.int32)

  pos_p = jnp.pad(positions.astype(f32), ((0, 0), (0, 13)))
  shifts_p = jnp.pad(shifts.astype(f32), ((0, 0), (0, 13)))
  na_p = jnp.pad(node_attrs.astype(f32), ((0, 0), (0, 6)))
  batch2d = batch.astype(jnp.int32).reshape(N, 1)
  zeros_nh = jnp.zeros((N, H), f32)

  wupe = jnp.pad(W_embed @ W_up[0], ((0, 6), (0, 0)))
  wsce = jnp.pad(W_embed @ W_sc[0], ((0, 6), (0, 0)))
  aerow = jnp.pad(atomic_energies.reshape(1, -1), ((0, 0), (0, 6)))
  wr0row = W_r0.reshape(1, H)
  wr1brow = W_r1b.reshape(1, 16)
  wv = jnp.pad(w_sh[:, 1:4] * SQRT3, ((0, 0), (0, 13)))
  w0 = w_sh[:, 0]

  pos_s, pos_r = _scg(pos_p, snd, rcv)
  rp0, rp1 = _edge_tc(pos_s, pos_r, shifts_p, R_W1[0], R_W2[0],
                      R_W1[1], R_W2[1], wv, w0)
  h0 = _h0(na_p, wupe)
  pagg0 = _scatter(h0, rp0, snd, rcv, zeros_nh)
  nf1, h1, e0 = _c0(pagg0, na_p, batch2d, W_out[0], wsce, wr0row, aerow,
                    W_up[1])
  pagg1 = _scatter(h1, rp1, snd, rcv, zeros_nh)
  e1 = _c1(pagg1, nf1, batch2d, W_out[1], W_sc[1], W_r1a, wr1brow)

  return (e0 + e1)[0, :NUM_GRAPHS]


# speedup vs baseline: 1.9330x; 1.9330x over previous
import functools

import jax
import jax.numpy as jnp
from jax import lax
from jax.experimental import pallas as pl
from jax.experimental.pallas import tpu as pltpu
from jax.experimental.pallas import tpu_sc as plsc

N = 10000
E = 320000
H = 128
NUM_BESSEL = 8
R_MAX = 5.0
NUM_GRAPHS = 32
AVG_NEIGH = 32.0
SQRT3 = 3.0 ** 0.5

NC = 2
NS = 16
NW = NC * NS
PER_W = E // NW
CHUNK = 80
NCHUNK = PER_W // CHUNK
NPAD = 10240
ROWS_PER_SUB = NPAD // NS

@functools.cache
def _sc_mesh():
  return plsc.VectorSubcoreMesh(core_axis_name="c", subcore_axis_name="s",
                                num_cores=NC, num_subcores=NS)


def _scg_body(pos_hbm, snd_hbm, rcv_hbm, out_s, out_r,
              sidx, ridx, bs, br, sem_a, sem_b):
  cid = lax.axis_index("c")
  sid = lax.axis_index("s")
  wid = cid * NS + sid

  def chunk(c, _):
    base = wid * PER_W + c * CHUNK
    pltpu.sync_copy(snd_hbm.at[pl.ds(base, CHUNK)], sidx)
    pltpu.sync_copy(rcv_hbm.at[pl.ds(base, CHUNK)], ridx)
    cp_a = pltpu.async_copy(pos_hbm.at[sidx], bs, sem_a)
    cp_b = pltpu.async_copy(pos_hbm.at[ridx], br, sem_b)
    cp_a.wait()
    cp_b.wait()
    pltpu.sync_copy(bs, out_s.at[pl.ds(base, CHUNK)])
    pltpu.sync_copy(br, out_r.at[pl.ds(base, CHUNK)])
    return 0

  lax.fori_loop(0, NCHUNK, chunk, 0)


@functools.cache
def _scg_kernel():
  return pl.kernel(
    _scg_body,
    out_type=(
        jax.ShapeDtypeStruct((E, 16), jnp.float32),
        jax.ShapeDtypeStruct((E, 16), jnp.float32),
    ),
    mesh=_sc_mesh(),
    scratch_types=[
        pltpu.VMEM((CHUNK,), jnp.int32),
        pltpu.VMEM((CHUNK,), jnp.int32),
        pltpu.VMEM((CHUNK, 16), jnp.float32),
        pltpu.VMEM((CHUNK, 16), jnp.float32),
        pltpu.SemaphoreType.DMA,
        pltpu.SemaphoreType.DMA,
    ],
    compiler_params=pltpu.CompilerParams(use_tc_tiling_on_sc=False),
  )


def _scg(pos_p, snd, rcv):
  return _scg_kernel()(pos_p, snd, rcv)


def _scatter_body(h_hbm, rp_hbm, snd_hbm, rcv_hbm, zeros_hbm, out_hbm,
                  sidx, ridx, hbuf, rbuf, sem_g, acc):
  cid = lax.axis_index("c")
  sid = lax.axis_index("s")
  wid = cid * NS + sid

  row0 = sid * ROWS_PER_SUB
  pltpu.sync_copy(zeros_hbm.at[pl.ds(row0, ROWS_PER_SUB)],
                  acc.at[pl.ds(row0, ROWS_PER_SUB)])
  plsc.subcore_barrier()

  def chunk(c, _):
    base = wid * PER_W + c * CHUNK
    pltpu.sync_copy(snd_hbm.at[pl.ds(base, CHUNK)], sidx)
    pltpu.sync_copy(rcv_hbm.at[pl.ds(base, CHUNK)], ridx)
    pltpu.async_copy(h_hbm.at[sidx], hbuf, sem_g).wait()
    pltpu.sync_copy(rp_hbm.at[pl.ds(base, CHUNK)], rbuf)

    def mul_row(i, _):
      for k in range(H // 16):
        sl = pl.ds(k * 16, 16)
        hbuf[i, sl] = hbuf[i, sl] * rbuf[i, sl]
      return 0

    lax.fori_loop(0, CHUNK, mul_row, 0)
    pltpu.sync_copy(hbuf, acc.at[ridx], add=True)
    return 0

  lax.fori_loop(0, NCHUNK, chunk, 0)
  plsc.subcore_barrier()
  pltpu.sync_copy(acc.at[pl.ds(row0, ROWS_PER_SUB)],
                  out_hbm.at[cid, pl.ds(row0, ROWS_PER_SUB)])


@functools.cache
def _scatter_kernel():
  return pl.kernel(
    _scatter_body,
    out_type=jax.ShapeDtypeStruct((NC, NPAD, H), jnp.float32),
    mesh=_sc_mesh(),
    scratch_types=[
        pltpu.VMEM((CHUNK,), jnp.int32),
        pltpu.VMEM((CHUNK,), jnp.int32),
        pltpu.VMEM((CHUNK, H), jnp.float32),
        pltpu.VMEM((CHUNK, H), jnp.float32),
        pltpu.SemaphoreType.DMA,
        pltpu.VMEM_SHARED((NPAD, H), jnp.float32),
    ],
  )


def _scatter(h, rp, snd, rcv, zeros_nh):
  return _scatter_kernel()(h, rp, snd, rcv, zeros_nh)


EBLK = 2000


def _edge_tc_body(ps_ref, pr_ref, sh_ref, w1a_ref, w2a_ref, w1b_ref, w2b_ref,
                  wv_ref, w0_ref, rp0_ref, rp1_ref):
  d = pr_ref[...] - ps_ref[...] + sh_ref[...]
  len2 = jnp.sum(d * d, axis=1, keepdims=True)
  ln = jnp.sqrt(len2)
  lc = jnp.maximum(ln, 1e-9)
  x = ln * (1.0 / R_MAX)
  env = 1.0 + x ** 5 * (-21.0 + x * (35.0 - 15.0 * x))
  cut = jnp.where(x < 1.0, env, 0.0)
  nn = (lax.broadcasted_iota(jnp.int32, (1, NUM_BESSEL), 1)
        .astype(jnp.float32) + 1.0) * (jnp.pi / R_MAX)
  pref = (2.0 / R_MAX) ** 0.5
  ef = jnp.sin(lc * nn) * ((pref * cut) / lc)

  inv_lc = 1.0 / lc

  def radial(w1, w2, row):
    u = jnp.dot(ef, w1, preferred_element_type=jnp.float32)
    a = u * jax.nn.sigmoid(u)
    r = jnp.dot(a, w2, preferred_element_type=jnp.float32)
    dotd = jnp.sum(d * wv_ref[row:row + 1, :], axis=1, keepdims=True)
    sh = w0_ref[row] + dotd * inv_lc
    return r * (sh * (1.0 / AVG_NEIGH))

  rp0_ref[...] = radial(w1a_ref[...], w2a_ref[...], 0)
  rp1_ref[...] = radial(w1b_ref[...], w2b_ref[...], 1)


def _edge_tc(ps, pr, shp, w1a, w2a, w1b, w2b, wv, w0):
  g = E // EBLK
  eb = lambda i: (i, 0)
  wfull = lambda i: (0, 0)
  return pl.pallas_call(
      _edge_tc_body,
      grid=(g,),
      in_specs=[
          pl.BlockSpec((EBLK, 16), eb),
          pl.BlockSpec((EBLK, 16), eb),
          pl.BlockSpec((EBLK, 16), eb),
          pl.BlockSpec((NUM_BESSEL, 64), wfull),
          pl.BlockSpec((64, H), wfull),
          pl.BlockSpec((NUM_BESSEL, 64), wfull),
          pl.BlockSpec((64, H), wfull),
          pl.BlockSpec((2, 16), wfull),
          pl.BlockSpec(memory_space=pltpu.SMEM),
      ],
      out_specs=[
          pl.BlockSpec((EBLK, H), eb),
          pl.BlockSpec((EBLK, H), eb),
      ],
      out_shape=[
          jax.ShapeDtypeStruct((E, H), jnp.float32),
          jax.ShapeDtypeStruct((E, H), jnp.float32),
      ],
  )(ps, pr, shp, w1a, w2a, w1b, w2b, wv, w0)


NBLK = 1000


def _h0_body(na_ref, w_ref, out_ref):
  out_ref[...] = jnp.dot(na_ref[...], w_ref[...],
                         preferred_element_type=jnp.float32)


def _h0(na_p, wupe):
  return pl.pallas_call(
      _h0_body,
      grid=(N // NBLK,),
      in_specs=[
          pl.BlockSpec((NBLK, 16), lambda i: (i, 0)),
          pl.BlockSpec((16, H), lambda i: (0, 0)),
      ],
      out_specs=pl.BlockSpec((NBLK, H), lambda i: (i, 0)),
      out_shape=jax.ShapeDtypeStruct((N, H), jnp.float32),
  )(na_p, wupe)


def _c0_body(pagg_ref, na_ref, batch_ref, wout_ref, wsce_ref, wr0_ref,
             ae_ref, wup1_ref, nf1_ref, h1_ref, e_ref):
  agg = pagg_ref[0] + pagg_ref[1]
  nf1 = jnp.dot(agg, wout_ref[...], preferred_element_type=jnp.float32) \
      + jnp.dot(na_ref[...], wsce_ref[...], preferred_element_type=jnp.float32)
  nf1_ref[...] = nf1
  h1_ref[...] = jnp.dot(nf1, wup1_ref[...], preferred_element_type=jnp.float32)
  en = jnp.sum(nf1 * wr0_ref[...], axis=1) \
      + jnp.sum(na_ref[...] * ae_ref[...], axis=1)
  gid = lax.broadcasted_iota(jnp.int32, (NBLK, H), 1)
  mask = batch_ref[...] == gid
  contrib = jnp.sum(jnp.where(mask, en[:, None], 0.0), axis=0)

  @pl.when(pl.program_id(0) == 0)
  def _():
    e_ref[...] = jnp.zeros_like(e_ref)

  e_ref[...] += contrib[None, :]


def _c0(pagg, na_p, batch2d, wout0, wsce, wr0row, aerow, wup1):
  nb = lambda i: (i, 0)
  wfull = lambda i: (0, 0)
  return pl.pallas_call(
      _c0_body,
      grid=(N // NBLK,),
      in_specs=[
          pl.BlockSpec((NC, NBLK, H), lambda i: (0, i, 0)),
          pl.BlockSpec((NBLK, 16), nb),
          pl.BlockSpec((NBLK, 1), nb),
          pl.BlockSpec((H, H), wfull),
          pl.BlockSpec((16, H), wfull),
          pl.BlockSpec((1, H), wfull),
          pl.BlockSpec((1, 16), wfull),
          pl.BlockSpec((H, H), wfull),
      ],
      out_specs=[
          pl.BlockSpec((NBLK, H), nb),
          pl.BlockSpec((NBLK, H), nb),
          pl.BlockSpec((1, H), wfull),
      ],
      out_shape=[
          jax.ShapeDtypeStruct((N, H), jnp.float32),
          jax.ShapeDtypeStruct((N, H), jnp.float32),
          jax.ShapeDtypeStruct((1, H), jnp.float32),
      ],
  )(pagg, na_p, batch2d, wout0, wsce, wr0row, aerow, wup1)


def _c1_body(pagg_ref, nf1_ref, batch_ref, wout_ref, wsc_ref, wr1a_ref,
             wr1b_ref, e_ref):
  agg = pagg_ref[0] + pagg_ref[1]
  nf2 = jnp.dot(agg, wout_ref[...], preferred_element_type=jnp.float32) \
      + jnp.dot(nf1_ref[...], wsc_ref[...], preferred_element_type=jnp.float32)
  t = jnp.dot(nf2, wr1a_ref[...], preferred_element_type=jnp.float32)
  t = t * jax.nn.sigmoid(t)
  en = jnp.sum(t * wr1b_ref[...], axis=1)
  gid = lax.broadcasted_iota(jnp.int32, (NBLK, H), 1)
  mask = batch_ref[...] == gid
  contrib = jnp.sum(jnp.where(mask, en[:, None], 0.0), axis=0)

  @pl.when(pl.program_id(0) == 0)
  def _():
    e_ref[...] = jnp.zeros_like(e_ref)

  e_ref[...] += contrib[None, :]


def _c1(pagg, nf1, batch2d, wout1, wsc1, wr1a, wr1brow):
  nb = lambda i: (i, 0)
  wfull = lambda i: (0, 0)
  return pl.pallas_call(
      _c1_body,
      grid=(N // NBLK,),
      in_specs=[
          pl.BlockSpec((NC, NBLK, H), lambda i: (0, i, 0)),
          pl.BlockSpec((NBLK, H), nb),
          pl.BlockSpec((NBLK, 1), nb),
          pl.BlockSpec((H, H), wfull),
          pl.BlockSpec((H, H), wfull),
          pl.BlockSpec((H, 16), wfull),
          pl.BlockSpec((1, 16), wfull),
      ],
      out_specs=pl.BlockSpec((1, H), wfull),
      out_shape=jax.ShapeDtypeStruct((1, H), jnp.float32),
  )(pagg, nf1, batch2d, wout1, wsc1, wr1a, wr1brow)


def kernel(positions, node_attrs, shifts, atomic_energies, W_embed, W_up,
           R_W1, R_W2, w_sh, W_out, W_sc, W_r0, W_r1a, W_r1b,
           edge_index, batch):
  f32 = jnp.float32
  snd = edge_index[0].astype(jnp.int32)
  rcv = edge_index[1].astype(jnp.int32)

  pos_p = jnp.pad(positions.astype(f32), ((0, 0), (0, 13)))
  shifts_p = jnp.pad(shifts.astype(f32), ((0, 0), (0, 13)))
  na_p = jnp.pad(node_attrs.astype(f32), ((0, 0), (0, 6)))
  batch2d = batch.astype(jnp.int32).reshape(N, 1)
  zeros_nh = jnp.zeros((NPAD, H), f32)

  wupe = jnp.pad(W_embed @ W_up[0], ((0, 6), (0, 0)))
  wsce = jnp.pad(W_embed @ W_sc[0], ((0, 6), (0, 0)))
  aerow = jnp.pad(atomic_energies.reshape(1, -1), ((0, 0), (0, 6)))
  wr0row = W_r0.reshape(1, H)
  wr1brow = W_r1b.reshape(1, 16)
  wv = jnp.pad(w_sh[:, 1:4] * SQRT3, ((0, 0), (0, 13)))
  w0 = w_sh[:, 0]

  pos_s, pos_r = _scg(pos_p, snd, rcv)
  rp0, rp1 = _edge_tc(pos_s, pos_r, shifts_p, R_W1[0], R_W2[0],
                      R_W1[1], R_W2[1], wv, w0)
  h0 = _h0(na_p, wupe)
  pagg0 = _scatter(h0, rp0, snd, rcv, zeros_nh)
  nf1, h1, e0 = _c0(pagg0, na_p, batch2d, W_out[0], wsce, wr0row, aerow,
                    W_up[1])
  pagg1 = _scatter(h1, rp1, snd, rcv, zeros_nh)
  e1 = _c1(pagg1, nf1, batch2d, W_out[1], W_sc[1], W_r1a, wr1brow)

  return (e0 + e1)[0, :NUM_GRAPHS]


# speedup vs baseline: 3.0439x; 1.5747x over previous
import functools

import jax
import jax.numpy as jnp
from jax import lax
from jax.experimental import pallas as pl
from jax.experimental.pallas import tpu as pltpu
from jax.experimental.pallas import tpu_sc as plsc

N = 10000
E = 320000
H = 128
NUM_BESSEL = 8
R_MAX = 5.0
NUM_GRAPHS = 32
AVG_NEIGH = 32.0
SQRT3 = 3.0 ** 0.5

NC = 2
NS = 16
NW = NC * NS
PER_W = E // NW
CHUNK = 80
NCHUNK = PER_W // CHUNK
SCHUNK = 40
SNCHUNK = PER_W // SCHUNK
SNCHUNK_PAD = 256
GROUP = 8
NPAD = 10240
ROWS_PER_SUB = NPAD // NS
NBUF = 3


@functools.cache
def _sc_mesh():
  return plsc.VectorSubcoreMesh(core_axis_name="c", subcore_axis_name="s",
                                num_cores=NC, num_subcores=NS)


def _pipeline(p_slot, s_slot, nchunk):
  s_slot(0, 0, True)
  s_slot(1, 1, True)
  p_slot(0, 0)
  s_slot(2, 2, True)

  full = (nchunk - 1) // 3

  def outer(i, _):
    c0 = 1 + i * 3
    for k in range(3):
      c = c0 + k
      b = (1 + k) % 3
      p_slot(c, b)
      bn = k
      @pl.when(c + 2 < nchunk)
      def _():
        s_slot(c + 2, bn, False)
    return 0

  lax.fori_loop(0, full, outer, 0)
  for c in range(3 * full + 1, nchunk):
    p_slot(c, c % 3)
    if c + 2 < nchunk:
      s_slot(c + 2, (c + 2) % 3, False)


def _scg_body(pos_hbm, snd3, rcv3, out_d,
              sall, rall, bs0, bs1, bs2, br0, br1, br2,
              gs0, gs1, gs2, gr0, gr1, gr2, so0, so1, so2):
  cid = lax.axis_index("c")
  sid = lax.axis_index("s")
  wid = cid * NS + sid
  base_w = wid * PER_W

  pltpu.sync_copy(snd3.at[wid], sall)
  pltpu.sync_copy(rcv3.at[wid], rall)

  bs = (bs0, bs1, bs2)
  br = (br0, br1, br2)
  gs = (gs0, gs1, gs2)
  gr = (gr0, gr1, gr2)
  so = (so0, so1, so2)

  def eoff(c):
    return pl.multiple_of(base_w + c * CHUNK, CHUNK)

  def s_slot(c, b, first):
    if not first:
      pltpu.make_async_copy(br[b], out_d.at[pl.ds(0, CHUNK)], so[b]).wait()
    pltpu.async_copy(pos_hbm.at[sall.at[c]], bs[b], gs[b])
    pltpu.async_copy(pos_hbm.at[rall.at[c]], br[b], gr[b])

  def p_slot(c, b):
    pltpu.make_async_copy(pos_hbm.at[sall.at[c]], bs[b], gs[b]).wait()
    pltpu.make_async_copy(pos_hbm.at[rall.at[c]], br[b], gr[b]).wait()
    bsb, brb = bs[b], br[b]

    def sub_row(i, _):
      brb[i, :] = brb[i, :] - bsb[i, :]
      return 0

    lax.fori_loop(0, CHUNK, sub_row, 0)
    pltpu.async_copy(brb, out_d.at[pl.ds(eoff(c), CHUNK)], so[b])

  _pipeline(p_slot, s_slot, NCHUNK)
  for b in range(NBUF):
    pltpu.make_async_copy(br[b], out_d.at[pl.ds(0, CHUNK)], so[b]).wait()


@functools.cache
def _scg_kernel():
  return pl.kernel(
    _scg_body,
    out_type=jax.ShapeDtypeStruct((E, 16), jnp.float32),
    mesh=_sc_mesh(),
    scratch_types=(
        [pltpu.VMEM((NCHUNK, CHUNK), jnp.int32)] * 2
        + [pltpu.VMEM((CHUNK, 16), jnp.float32)] * 6
        + [pltpu.SemaphoreType.DMA] * 9
    ),
    compiler_params=pltpu.CompilerParams(use_tc_tiling_on_sc=False),
  )


def _scg(pos_p, snd3, rcv3):
  return _scg_kernel()(pos_p, snd3, rcv3)


def _scatter_body_simple(h_hbm, rp_hbm, snd_hbm, rcv_hbm, zeros_hbm, out_hbm,
                         sidx, ridx, hbuf, rbuf, sem_g, acc):
  cid = lax.axis_index("c")
  sid = lax.axis_index("s")
  wid = cid * NS + sid

  row0 = sid * ROWS_PER_SUB
  pltpu.sync_copy(zeros_hbm.at[pl.ds(row0, ROWS_PER_SUB)],
                  acc.at[pl.ds(row0, ROWS_PER_SUB)])
  plsc.subcore_barrier()

  def chunk(c, _):
    base = wid * PER_W + c * CHUNK
    pltpu.sync_copy(snd_hbm.at[pl.ds(base, CHUNK)], sidx)
    pltpu.sync_copy(rcv_hbm.at[pl.ds(base, CHUNK)], ridx)
    pltpu.async_copy(h_hbm.at[sidx], hbuf, sem_g).wait()
    pltpu.sync_copy(rp_hbm.at[pl.ds(base, CHUNK)], rbuf)

    def mul_row(i, _):
      for k in range(H // 16):
        sl = pl.ds(k * 16, 16)
        hbuf[i, sl] = hbuf[i, sl] * rbuf[i, sl]
      return 0

    lax.fori_loop(0, CHUNK, mul_row, 0)
    pltpu.sync_copy(hbuf, acc.at[ridx], add=True)
    return 0

  lax.fori_loop(0, NCHUNK, chunk, 0)
  plsc.subcore_barrier()
  pltpu.sync_copy(acc.at[pl.ds(row0, ROWS_PER_SUB)],
                  out_hbm.at[cid, pl.ds(row0, ROWS_PER_SUB)])


@functools.cache
def _scatter_simple_kernel():
  return pl.kernel(
    _scatter_body_simple,
    out_type=jax.ShapeDtypeStruct((NC, NPAD, H), jnp.float32),
    mesh=_sc_mesh(),
    scratch_types=(
        [pltpu.VMEM((CHUNK,), jnp.int32)] * 2
        + [pltpu.VMEM((CHUNK, H), jnp.float32)] * 2
        + [pltpu.SemaphoreType.DMA]
        + [pltpu.VMEM_SHARED((NPAD, H), jnp.float32)]
    ),
  )


def _scatter_body(h_hbm, rp_hbm, snd3, rcv3, zeros_hbm, out_hbm,
                  sidx2, ridx2, hb0, hb1, hb2, rb0, rb1, rb2,
                  sg0, sg1, sg2, sr0, sr1, sr2, ss0, ss1, ss2, acc):
  cid = lax.axis_index("c")
  sid = lax.axis_index("s")
  wid = cid * NS + sid
  base_w = wid * PER_W

  row0 = sid * ROWS_PER_SUB
  pltpu.sync_copy(zeros_hbm.at[pl.ds(row0, ROWS_PER_SUB)],
                  acc.at[pl.ds(row0, ROWS_PER_SUB)])
  pltpu.sync_copy(snd3.at[wid, pl.ds(0, 2 * GROUP)], sidx2)
  pltpu.sync_copy(rcv3.at[wid, pl.ds(0, 2 * GROUP)], ridx2)
  plsc.subcore_barrier()

  hb = (hb0, hb1, hb2)
  rb = (rb0, rb1, rb2)
  sg = (sg0, sg1, sg2)
  sr = (sr0, sr1, sr2)
  ss = (ss0, ss1, ss2)

  def eoff(c):
    return pl.multiple_of(base_w + c * SCHUNK, SCHUNK)

  def s_slot(c, b, first):
    if not first:
      pltpu.make_async_copy(hb[b], acc.at[pl.ds(0, SCHUNK)], ss[b]).wait()
    pltpu.async_copy(h_hbm.at[sidx2.at[lax.rem(c, 2 * GROUP)]], hb[b], sg[b])
    pltpu.async_copy(rp_hbm.at[pl.ds(eoff(c), SCHUNK)], rb[b], sr[b])

  def p_slot(c, b):
    pltpu.make_async_copy(h_hbm.at[sidx2.at[0]], hb[b], sg[b]).wait()
    pltpu.make_async_copy(rp_hbm.at[pl.ds(0, SCHUNK)], rb[b], sr[b]).wait()
    hbb, rbb = hb[b], rb[b]

    def mul_row(i, _):
      for k in range(H // 16):
        sl = pl.ds(k * 16, 16)
        hbb[i, sl] = hbb[i, sl] * rbb[i, sl]
      return 0

    lax.fori_loop(0, SCHUNK, mul_row, 0)
    pltpu.async_copy(hbb, acc.at[ridx2.at[lax.rem(c, 2 * GROUP)]], ss[b],
                     add=True)
    c = jnp.int32(c)
    g1 = lax.div(c, jnp.int32(GROUP)) + 1

    @pl.when((lax.rem(c, jnp.int32(GROUP)) == 4) & (c >= GROUP)
             & (g1 < SNCHUNK_PAD // GROUP))
    def _():
      half = lax.rem(g1, 2) * GROUP
      pltpu.sync_copy(snd3.at[wid, pl.ds(g1 * GROUP, GROUP)],
                      sidx2.at[pl.ds(half, GROUP)])
      pltpu.sync_copy(rcv3.at[wid, pl.ds(g1 * GROUP, GROUP)],
                      ridx2.at[pl.ds(half, GROUP)])

  _pipeline(p_slot, s_slot, SNCHUNK)
  for b in range(NBUF):
    pltpu.make_async_copy(hb[b], acc.at[pl.ds(0, SCHUNK)], ss[b]).wait()

  plsc.subcore_barrier()
  pltpu.sync_copy(acc.at[pl.ds(row0, ROWS_PER_SUB)],
                  out_hbm.at[cid, pl.ds(row0, ROWS_PER_SUB)])


@functools.cache
def _scatter_kernel():
  return pl.kernel(
    _scatter_body,
    out_type=jax.ShapeDtypeStruct((NC, NPAD, H), jnp.float32),
    mesh=_sc_mesh(),
    scratch_types=(
        [pltpu.VMEM((2 * GROUP, SCHUNK), jnp.int32)] * 2
        + [pltpu.VMEM((SCHUNK, H), jnp.float32)] * 6
        + [pltpu.SemaphoreType.DMA] * 9
        + [pltpu.VMEM_SHARED((NPAD, H), jnp.float32)]
    ),
  )


def _scatter(h, rp, snd3, rcv3, zeros_nh):
  return _scatter_kernel()(h, rp, snd3, rcv3, zeros_nh)


def _scatter_s(h, rp, snd, rcv, zeros_nh):
  return _scatter_simple_kernel()(h, rp, snd, rcv, zeros_nh)


EBLK = 2000


def _edge_tc_body(d_ref, sh_ref, w1a_ref, w2a_ref, w1b_ref, w2b_ref,
                  wv_ref, w0_ref, rp0_ref, rp1_ref):
  d = d_ref[:, :3] + sh_ref[...]
  len2 = jnp.sum(d * d, axis=1, keepdims=True)
  ln = jnp.sqrt(len2)
  lc = jnp.maximum(ln, 1e-9)
  x = ln * (1.0 / R_MAX)
  env = 1.0 + x ** 5 * (-21.0 + x * (35.0 - 15.0 * x))
  cut = jnp.where(x < 1.0, env, 0.0)
  nn = (lax.broadcasted_iota(jnp.int32, (1, NUM_BESSEL), 1)
        .astype(jnp.float32) + 1.0) * (jnp.pi / R_MAX)
  pref = (2.0 / R_MAX) ** 0.5
  ef = jnp.sin(lc * nn) * ((pref * cut) / lc)

  inv_lc = 1.0 / lc

  def radial(w1, w2, row):
    u = jnp.dot(ef, w1, preferred_element_type=jnp.float32)
    a = u * jax.nn.sigmoid(u)
    r = jnp.dot(a, w2, preferred_element_type=jnp.float32)
    dotd = jnp.sum(d * wv_ref[row:row + 1, :], axis=1, keepdims=True)
    sh = w0_ref[row] + dotd * inv_lc
    return r * (sh * (1.0 / AVG_NEIGH))

  rp0_ref[...] = radial(w1a_ref[...], w2a_ref[...], 0)
  rp1_ref[...] = radial(w1b_ref[...], w2b_ref[...], 1)


def _edge_tc(d_lin, shifts, w1a, w2a, w1b, w2b, wv, w0):
  g = E // EBLK
  eb = lambda i: (i, 0)
  wfull = lambda i: (0, 0)
  return pl.pallas_call(
      _edge_tc_body,
      grid=(g,),
      in_specs=[
          pl.BlockSpec((EBLK, 16), eb),
          pl.BlockSpec((EBLK, 3), eb),
          pl.BlockSpec((NUM_BESSEL, 64), wfull),
          pl.BlockSpec((64, H), wfull),
          pl.BlockSpec((NUM_BESSEL, 64), wfull),
          pl.BlockSpec((64, H), wfull),
          pl.BlockSpec((2, 3), wfull),
          pl.BlockSpec(memory_space=pltpu.SMEM),
      ],
      out_specs=[
          pl.BlockSpec((EBLK, H), eb),
          pl.BlockSpec((EBLK, H), eb),
      ],
      out_shape=[
          jax.ShapeDtypeStruct((E, H), jnp.float32),
          jax.ShapeDtypeStruct((E, H), jnp.float32),
      ],
  )(d_lin, shifts, w1a, w2a, w1b, w2b, wv, w0)


NBLK = 1000


def _h0_body(na_ref, w_ref, out_ref):
  out_ref[...] = jnp.dot(na_ref[...], w_ref[...],
                         preferred_element_type=jnp.float32)


def _h0(na_p, wupe):
  return pl.pallas_call(
      _h0_body,
      grid=(N // NBLK,),
      in_specs=[
          pl.BlockSpec((NBLK, 16), lambda i: (i, 0)),
          pl.BlockSpec((16, H), lambda i: (0, 0)),
      ],
      out_specs=pl.BlockSpec((NBLK, H), lambda i: (i, 0)),
      out_shape=jax.ShapeDtypeStruct((N, H), jnp.float32),
  )(na_p, wupe)


def _c0_body(pagg_ref, na_ref, batch_ref, wout_ref, wsce_ref, wr0_ref,
             ae_ref, wup1_ref, nf1_ref, h1_ref, e_ref):
  agg = pagg_ref[0] + pagg_ref[1]
  nf1 = jnp.dot(agg, wout_ref[...], preferred_element_type=jnp.float32) \
      + jnp.dot(na_ref[...], wsce_ref[...], preferred_element_type=jnp.float32)
  nf1_ref[...] = nf1
  h1_ref[...] = jnp.dot(nf1, wup1_ref[...], preferred_element_type=jnp.float32)
  en = jnp.sum(nf1 * wr0_ref[...], axis=1) \
      + jnp.sum(na_ref[...] * ae_ref[...], axis=1)
  gid = lax.broadcasted_iota(jnp.int32, (NBLK, H), 1)
  mask = batch_ref[...] == gid
  contrib = jnp.sum(jnp.where(mask, en[:, None], 0.0), axis=0)

  @pl.when(pl.program_id(0) == 0)
  def _():
    e_ref[...] = jnp.zeros_like(e_ref)

  e_ref[...] += contrib[None, :]


def _c0(pagg, na_p, batch2d, wout0, wsce, wr0row, aerow, wup1):
  nb = lambda i: (i, 0)
  wfull = lambda i: (0, 0)
  return pl.pallas_call(
      _c0_body,
      grid=(N // NBLK,),
      in_specs=[
          pl.BlockSpec((NC, NBLK, H), lambda i: (0, i, 0)),
          pl.BlockSpec((NBLK, 16), nb),
          pl.BlockSpec((NBLK, 1), nb),
          pl.BlockSpec((H, H), wfull),
          pl.BlockSpec((16, H), wfull),
          pl.BlockSpec((1, H), wfull),
          pl.BlockSpec((1, 16), wfull),
          pl.BlockSpec((H, H), wfull),
      ],
      out_specs=[
          pl.BlockSpec((NBLK, H), nb),
          pl.BlockSpec((NBLK, H), nb),
          pl.BlockSpec((1, H), wfull),
      ],
      out_shape=[
          jax.ShapeDtypeStruct((N, H), jnp.float32),
          jax.ShapeDtypeStruct((N, H), jnp.float32),
          jax.ShapeDtypeStruct((1, H), jnp.float32),
      ],
  )(pagg, na_p, batch2d, wout0, wsce, wr0row, aerow, wup1)


def _c1_body(pagg_ref, nf1_ref, batch_ref, wout_ref, wsc_ref, wr1a_ref,
             wr1b_ref, e_ref):
  agg = pagg_ref[0] + pagg_ref[1]
  nf2 = jnp.dot(agg, wout_ref[...], preferred_element_type=jnp.float32) \
      + jnp.dot(nf1_ref[...], wsc_ref[...], preferred_element_type=jnp.float32)
  t = jnp.dot(nf2, wr1a_ref[...], preferred_element_type=jnp.float32)
  t = t * jax.nn.sigmoid(t)
  en = jnp.sum(t * wr1b_ref[...], axis=1)
  gid = lax.broadcasted_iota(jnp.int32, (NBLK, H), 1)
  mask = batch_ref[...] == gid
  contrib = jnp.sum(jnp.where(mask, en[:, None], 0.0), axis=0)

  @pl.when(pl.program_id(0) == 0)
  def _():
    e_ref[...] = jnp.zeros_like(e_ref)

  e_ref[...] += contrib[None, :]


def _c1(pagg, nf1, batch2d, wout1, wsc1, wr1a, wr1brow):
  nb = lambda i: (i, 0)
  wfull = lambda i: (0, 0)
  return pl.pallas_call(
      _c1_body,
      grid=(N // NBLK,),
      in_specs=[
          pl.BlockSpec((NC, NBLK, H), lambda i: (0, i, 0)),
          pl.BlockSpec((NBLK, H), nb),
          pl.BlockSpec((NBLK, 1), nb),
          pl.BlockSpec((H, H), wfull),
          pl.BlockSpec((H, H), wfull),
          pl.BlockSpec((H, 16), wfull),
          pl.BlockSpec((1, 16), wfull),
      ],
      out_specs=pl.BlockSpec((1, H), wfull),
      out_shape=jax.ShapeDtypeStruct((1, H), jnp.float32),
  )(pagg, nf1, batch2d, wout1, wsc1, wr1a, wr1brow)


def kernel(positions, node_attrs, shifts, atomic_energies, W_embed, W_up,
           R_W1, R_W2, w_sh, W_out, W_sc, W_r0, W_r1a, W_r1b,
           edge_index, batch):
  f32 = jnp.float32
  snd = edge_index[0].astype(jnp.int32)
  rcv = edge_index[1].astype(jnp.int32)
  snd3 = snd.reshape(NW, NCHUNK, CHUNK)
  rcv3 = rcv.reshape(NW, NCHUNK, CHUNK)
  pad_c = ((0, 0), (0, SNCHUNK_PAD - SNCHUNK), (0, 0))
  snd3s = jnp.pad(snd.reshape(NW, SNCHUNK, SCHUNK), pad_c)
  rcv3s = jnp.pad(rcv.reshape(NW, SNCHUNK, SCHUNK), pad_c)

  pos_p = jnp.pad(positions.astype(f32), ((0, 0), (0, 13)))
  na_p = jnp.pad(node_attrs.astype(f32), ((0, 0), (0, 6)))
  batch2d = batch.astype(jnp.int32).reshape(N, 1)
  zeros_nh = jnp.zeros((NPAD, H), f32)

  wupe = jnp.pad(W_embed @ W_up[0], ((0, 6), (0, 0)))
  wsce = jnp.pad(W_embed @ W_sc[0], ((0, 6), (0, 0)))
  aerow = jnp.pad(atomic_energies.reshape(1, -1), ((0, 0), (0, 6)))
  wr0row = W_r0.reshape(1, H)
  wr1brow = W_r1b.reshape(1, 16)
  wv = w_sh[:, 1:4] * SQRT3
  w0 = w_sh[:, 0]

  d_lin = _scg(pos_p, snd3, rcv3)
  rp0, rp1 = _edge_tc(d_lin, shifts.astype(f32), R_W1[0], R_W2[0],
                      R_W1[1], R_W2[1], wv, w0)
  h0 = _h0(na_p, wupe)
  pagg0 = _scatter(h0, rp0, snd3s, rcv3s, zeros_nh)
  nf1, h1, e0 = _c0(pagg0, na_p, batch2d, W_out[0], wsce, wr0row, aerow,
                    W_up[1])
  pagg1 = _scatter(h1, rp1, snd3s, rcv3s, zeros_nh)
  e1 = _c1(pagg1, nf1, batch2d, W_out[1], W_sc[1], W_r1a, wr1brow)

  return (e0 + e1)[0, :NUM_GRAPHS]


# speedup vs baseline: 4.3613x; 1.4328x over previous
import functools

import jax
import jax.numpy as jnp
from jax import lax
from jax.experimental import pallas as pl
from jax.experimental.pallas import tpu as pltpu
from jax.experimental.pallas import tpu_sc as plsc

N = 10000
E = 320000
H = 128
NUM_BESSEL = 8
R_MAX = 5.0
NUM_GRAPHS = 32
AVG_NEIGH = 32.0
SQRT3 = 3.0 ** 0.5

NC = 2
NS = 16
NW = NC * NS
PER_W = E // NW
CHUNK = 80
NCHUNK = PER_W // CHUNK
SCHUNK = 40
SNCHUNK = PER_W // SCHUNK
SNCHUNK_PAD = 256
GROUP = 8
NPAD = 10240
ROWS_PER_SUB = NPAD // NS
NBUF = 3


@functools.cache
def _sc_mesh():
  return plsc.VectorSubcoreMesh(core_axis_name="c", subcore_axis_name="s",
                                num_cores=NC, num_subcores=NS)


def _pipeline(p_slot, s_slot, nchunk):
  s_slot(0, 0, True)
  s_slot(1, 1, True)
  p_slot(0, 0)
  s_slot(2, 2, True)

  full = (nchunk - 1) // 3

  def outer(i, _):
    c0 = 1 + i * 3
    for k in range(3):
      c = c0 + k
      b = (1 + k) % 3
      p_slot(c, b)
      bn = k
      @pl.when(c + 2 < nchunk)
      def _():
        s_slot(c + 2, bn, False)
    return 0

  lax.fori_loop(0, full, outer, 0)
  for c in range(3 * full + 1, nchunk):
    p_slot(c, c % 3)
    if c + 2 < nchunk:
      s_slot(c + 2, (c + 2) % 3, False)


def _scg_body(pos_hbm, snd3, rcv3, out_d,
              sall, rall, bs0, bs1, bs2, br0, br1, br2,
              gs0, gs1, gs2, gr0, gr1, gr2, so0, so1, so2):
  cid = lax.axis_index("c")
  sid = lax.axis_index("s")
  wid = cid * NS + sid
  base_w = wid * PER_W

  pltpu.sync_copy(snd3.at[wid], sall)
  pltpu.sync_copy(rcv3.at[wid], rall)

  bs = (bs0, bs1, bs2)
  br = (br0, br1, br2)
  gs = (gs0, gs1, gs2)
  gr = (gr0, gr1, gr2)
  so = (so0, so1, so2)

  def eoff(c):
    return pl.multiple_of(base_w + c * CHUNK, CHUNK)

  def s_slot(c, b, first):
    if not first:
      pltpu.make_async_copy(br[b], out_d.at[pl.ds(0, CHUNK)], so[b]).wait()
    pltpu.async_copy(pos_hbm.at[sall.at[c]], bs[b], gs[b])
    pltpu.async_copy(pos_hbm.at[rall.at[c]], br[b], gr[b])

  def p_slot(c, b):
    pltpu.make_async_copy(pos_hbm.at[sall.at[c]], bs[b], gs[b]).wait()
    pltpu.make_async_copy(pos_hbm.at[rall.at[c]], br[b], gr[b]).wait()
    bsb, brb = bs[b], br[b]

    def sub_row(i, _):
      brb[i, :] = brb[i, :] - bsb[i, :]
      return 0

    lax.fori_loop(0, CHUNK, sub_row, 0)
    pltpu.async_copy(brb, out_d.at[pl.ds(eoff(c), CHUNK)], so[b])

  _pipeline(p_slot, s_slot, NCHUNK)
  for b in range(NBUF):
    pltpu.make_async_copy(br[b], out_d.at[pl.ds(0, CHUNK)], so[b]).wait()


@functools.cache
def _scg_kernel():
  return pl.kernel(
    _scg_body,
    out_type=jax.ShapeDtypeStruct((E, 16), jnp.float32),
    mesh=_sc_mesh(),
    scratch_types=(
        [pltpu.VMEM((NCHUNK, CHUNK), jnp.int32)] * 2
        + [pltpu.VMEM((CHUNK, 16), jnp.float32)] * 6
        + [pltpu.SemaphoreType.DMA] * 9
    ),
    compiler_params=pltpu.CompilerParams(use_tc_tiling_on_sc=False),
  )


def _scg(pos_p, snd3, rcv3):
  return _scg_kernel()(pos_p, snd3, rcv3)


def _scatter_body_simple(h_hbm, rp_hbm, snd_hbm, rcv_hbm, zeros_hbm, out_hbm,
                         sidx, ridx, hbuf, rbuf, sem_g, acc):
  cid = lax.axis_index("c")
  sid = lax.axis_index("s")
  wid = cid * NS + sid

  row0 = sid * ROWS_PER_SUB
  pltpu.sync_copy(zeros_hbm.at[pl.ds(row0, ROWS_PER_SUB)],
                  acc.at[pl.ds(row0, ROWS_PER_SUB)])
  plsc.subcore_barrier()

  def chunk(c, _):
    base = wid * PER_W + c * CHUNK
    pltpu.sync_copy(snd_hbm.at[pl.ds(base, CHUNK)], sidx)
    pltpu.sync_copy(rcv_hbm.at[pl.ds(base, CHUNK)], ridx)
    pltpu.async_copy(h_hbm.at[sidx], hbuf, sem_g).wait()
    pltpu.sync_copy(rp_hbm.at[pl.ds(base, CHUNK)], rbuf)

    def mul_row(i, _):
      for k in range(H // 16):
        sl = pl.ds(k * 16, 16)
        hbuf[i, sl] = hbuf[i, sl] * rbuf[i, sl]
      return 0

    lax.fori_loop(0, CHUNK, mul_row, 0)
    pltpu.sync_copy(hbuf, acc.at[ridx], add=True)
    return 0

  lax.fori_loop(0, NCHUNK, chunk, 0)
  plsc.subcore_barrier()
  pltpu.sync_copy(acc.at[pl.ds(row0, ROWS_PER_SUB)],
                  out_hbm.at[cid, pl.ds(row0, ROWS_PER_SUB)])


@functools.cache
def _scatter_simple_kernel():
  return pl.kernel(
    _scatter_body_simple,
    out_type=jax.ShapeDtypeStruct((NC, NPAD, H), jnp.float32),
    mesh=_sc_mesh(),
    scratch_types=(
        [pltpu.VMEM((CHUNK,), jnp.int32)] * 2
        + [pltpu.VMEM((CHUNK, H), jnp.float32)] * 2
        + [pltpu.SemaphoreType.DMA]
        + [pltpu.VMEM_SHARED((NPAD, H), jnp.float32)]
    ),
  )


def _scatter_body(h_hbm, rp_hbm, snd3, rcv3, zeros_hbm, out_hbm,
                  sidx2, ridx2, hb0, hb1, hb2, rb0, rb1, rb2,
                  sg0, sg1, sg2, sr0, sr1, sr2, ss0, ss1, ss2, acc):
  cid = lax.axis_index("c")
  sid = lax.axis_index("s")
  wid = cid * NS + sid
  base_w = wid * PER_W

  row0 = sid * ROWS_PER_SUB
  pltpu.sync_copy(zeros_hbm.at[pl.ds(row0, ROWS_PER_SUB)],
                  acc.at[pl.ds(row0, ROWS_PER_SUB)])
  pltpu.sync_copy(snd3.at[wid, pl.ds(0, 2 * GROUP)], sidx2)
  pltpu.sync_copy(rcv3.at[wid, pl.ds(0, 2 * GROUP)], ridx2)
  plsc.subcore_barrier()

  hb = (hb0, hb1, hb2)
  rb = (rb0, rb1, rb2)
  sg = (sg0, sg1, sg2)
  sr = (sr0, sr1, sr2)
  ss = (ss0, ss1, ss2)

  def eoff(c):
    return pl.multiple_of(base_w + c * SCHUNK, SCHUNK)

  def s_slot(c, b, first):
    if not first:
      pltpu.make_async_copy(hb[b], acc.at[pl.ds(0, SCHUNK)], ss[b]).wait()
    pltpu.async_copy(h_hbm.at[sidx2.at[lax.rem(c, 2 * GROUP)]], hb[b], sg[b])
    pltpu.async_copy(rp_hbm.at[pl.ds(eoff(c), SCHUNK)], rb[b], sr[b])

  def p_slot(c, b):
    pltpu.make_async_copy(h_hbm.at[sidx2.at[0]], hb[b], sg[b]).wait()
    pltpu.make_async_copy(rp_hbm.at[pl.ds(0, SCHUNK)], rb[b], sr[b]).wait()
    hbb, rbb = hb[b], rb[b]

    def mul_row(i, _):
      for k in range(H // 16):
        sl = pl.ds(k * 16, 16)
        hbb[i, sl] = hbb[i, sl] * rbb[i, sl]
      return 0

    lax.fori_loop(0, SCHUNK, mul_row, 0)
    pltpu.async_copy(hbb, acc.at[ridx2.at[lax.rem(c, 2 * GROUP)]], ss[b],
                     add=True)
    c = jnp.int32(c)
    g1 = lax.div(c, jnp.int32(GROUP)) + 1

    @pl.when((lax.rem(c, jnp.int32(GROUP)) == 4) & (c >= GROUP)
             & (g1 < SNCHUNK_PAD // GROUP))
    def _():
      half = lax.rem(g1, 2) * GROUP
      pltpu.sync_copy(snd3.at[wid, pl.ds(g1 * GROUP, GROUP)],
                      sidx2.at[pl.ds(half, GROUP)])
      pltpu.sync_copy(rcv3.at[wid, pl.ds(g1 * GROUP, GROUP)],
                      ridx2.at[pl.ds(half, GROUP)])

  _pipeline(p_slot, s_slot, SNCHUNK)
  for b in range(NBUF):
    pltpu.make_async_copy(hb[b], acc.at[pl.ds(0, SCHUNK)], ss[b]).wait()

  plsc.subcore_barrier()
  pltpu.sync_copy(acc.at[pl.ds(row0, ROWS_PER_SUB)],
                  out_hbm.at[cid, pl.ds(row0, ROWS_PER_SUB)])


@functools.cache
def _scatter_kernel():
  return pl.kernel(
    _scatter_body,
    out_type=jax.ShapeDtypeStruct((NC, NPAD, H), jnp.float32),
    mesh=_sc_mesh(),
    scratch_types=(
        [pltpu.VMEM((2 * GROUP, SCHUNK), jnp.int32)] * 2
        + [pltpu.VMEM((SCHUNK, H), jnp.float32)] * 6
        + [pltpu.SemaphoreType.DMA] * 9
        + [pltpu.VMEM_SHARED((NPAD, H), jnp.float32)]
    ),
  )


def _scatter(h, rp, snd3, rcv3, zeros_nh):
  return _scatter_kernel()(h, rp, snd3, rcv3, zeros_nh)


def _scatter_s(h, rp, snd, rcv, zeros_nh):
  return _scatter_simple_kernel()(h, rp, snd, rcv, zeros_nh)


EBLK = 2000


def _edge_tc_body(d_ref, sh_ref, w1a_ref, w2a_ref, w1b_ref, w2b_ref,
                  wv_ref, w0_ref, rp0_ref, rp1_ref):
  d = d_ref[:, :3] + sh_ref[...]
  len2 = jnp.sum(d * d, axis=1, keepdims=True)
  ln = jnp.sqrt(len2)
  lc = jnp.maximum(ln, 1e-9)
  x = ln * (1.0 / R_MAX)
  env = 1.0 + x ** 5 * (-21.0 + x * (35.0 - 15.0 * x))
  cut = jnp.where(x < 1.0, env, 0.0)
  nn_col = (lax.broadcasted_iota(jnp.int32, (NUM_BESSEL, 1), 0)
            .astype(jnp.float32) + 1.0) * (jnp.pi / R_MAX)
  pref = (2.0 / R_MAX) ** 0.5
  arg_t = nn_col * lax.transpose(lc, (1, 0))
  s = lax.transpose(jnp.sin(arg_t), (1, 0))
  ef = s * ((pref * cut) / lc)

  inv_lc = 1.0 / lc

  def radial(w1, w2, row):
    u = jnp.dot(ef, w1, preferred_element_type=jnp.float32)
    a = u * jax.nn.sigmoid(u)
    r = jnp.dot(a, w2, preferred_element_type=jnp.float32)
    dotd = jnp.sum(d * wv_ref[row:row + 1, :], axis=1, keepdims=True)
    sh = w0_ref[row] + dotd * inv_lc
    return r * (sh * (1.0 / AVG_NEIGH))

  rp0_ref[...] = radial(w1a_ref[...], w2a_ref[...], 0)
  rp1_ref[...] = radial(w1b_ref[...], w2b_ref[...], 1)


def _edge_tc(d_lin, shifts, w1a, w2a, w1b, w2b, wv, w0):
  g = E // EBLK
  eb = lambda i: (i, 0)
  wfull = lambda i: (0, 0)
  return pl.pallas_call(
      _edge_tc_body,
      grid=(g,),
      in_specs=[
          pl.BlockSpec((EBLK, 16), eb),
          pl.BlockSpec((EBLK, 3), eb),
          pl.BlockSpec((NUM_BESSEL, 64), wfull),
          pl.BlockSpec((64, H), wfull),
          pl.BlockSpec((NUM_BESSEL, 64), wfull),
          pl.BlockSpec((64, H), wfull),
          pl.BlockSpec((2, 3), wfull),
          pl.BlockSpec(memory_space=pltpu.SMEM),
      ],
      out_specs=[
          pl.BlockSpec((EBLK, H), eb),
          pl.BlockSpec((EBLK, H), eb),
      ],
      out_shape=[
          jax.ShapeDtypeStruct((E, H), jnp.float32),
          jax.ShapeDtypeStruct((E, H), jnp.float32),
      ],
  )(d_lin, shifts, w1a, w2a, w1b, w2b, wv, w0)


NBLK = 1000


def _h0_body(na_ref, w_ref, out_ref):
  out_ref[...] = jnp.dot(na_ref[...], w_ref[...],
                         preferred_element_type=jnp.float32)


def _h0(na_p, wupe):
  return pl.pallas_call(
      _h0_body,
      grid=(N // NBLK,),
      in_specs=[
          pl.BlockSpec((NBLK, 16), lambda i: (i, 0)),
          pl.BlockSpec((16, H), lambda i: (0, 0)),
      ],
      out_specs=pl.BlockSpec((NBLK, H), lambda i: (i, 0)),
      out_shape=jax.ShapeDtypeStruct((N, H), jnp.float32),
  )(na_p, wupe)


def _c0_body(pagg_ref, na_ref, batch_ref, wout_ref, wsce_ref, wr0_ref,
             ae_ref, wup1_ref, nf1_ref, h1_ref, e_ref):
  agg = pagg_ref[0] + pagg_ref[1]
  nf1 = jnp.dot(agg, wout_ref[...], preferred_element_type=jnp.float32) \
      + jnp.dot(na_ref[...], wsce_ref[...], preferred_element_type=jnp.float32)
  nf1_ref[...] = nf1
  h1_ref[...] = jnp.dot(nf1, wup1_ref[...], preferred_element_type=jnp.float32)
  en = jnp.sum(nf1 * wr0_ref[...], axis=1) \
      + jnp.sum(na_ref[...] * ae_ref[...], axis=1)
  gid = lax.broadcasted_iota(jnp.int32, (NBLK, H), 1)
  mask = batch_ref[...] == gid
  contrib = jnp.sum(jnp.where(mask, en[:, None], 0.0), axis=0)

  @pl.when(pl.program_id(0) == 0)
  def _():
    e_ref[...] = jnp.zeros_like(e_ref)

  e_ref[...] += contrib[None, :]


def _c0(pagg, na_p, batch2d, wout0, wsce, wr0row, aerow, wup1):
  nb = lambda i: (i, 0)
  wfull = lambda i: (0, 0)
  return pl.pallas_call(
      _c0_body,
      grid=(N // NBLK,),
      in_specs=[
          pl.BlockSpec((NC, NBLK, H), lambda i: (0, i, 0)),
          pl.BlockSpec((NBLK, 16), nb),
          pl.BlockSpec((NBLK, 1), nb),
          pl.BlockSpec((H, H), wfull),
          pl.BlockSpec((16, H), wfull),
          pl.BlockSpec((1, H), wfull),
          pl.BlockSpec((1, 16), wfull),
          pl.BlockSpec((H, H), wfull),
      ],
      out_specs=[
          pl.BlockSpec((NBLK, H), nb),
          pl.BlockSpec((NBLK, H), nb),
          pl.BlockSpec((1, H), wfull),
      ],
      out_shape=[
          jax.ShapeDtypeStruct((N, H), jnp.float32),
          jax.ShapeDtypeStruct((N, H), jnp.float32),
          jax.ShapeDtypeStruct((1, H), jnp.float32),
      ],
  )(pagg, na_p, batch2d, wout0, wsce, wr0row, aerow, wup1)


def _c1_body(pagg_ref, nf1_ref, batch_ref, wout_ref, wsc_ref, wr1a_ref,
             wr1b_ref, e_ref):
  agg = pagg_ref[0] + pagg_ref[1]
  nf2 = jnp.dot(agg, wout_ref[...], preferred_element_type=jnp.float32) \
      + jnp.dot(nf1_ref[...], wsc_ref[...], preferred_element_type=jnp.float32)
  t = jnp.dot(nf2, wr1a_ref[...], preferred_element_type=jnp.float32)
  t = t * jax.nn.sigmoid(t)
  en = jnp.sum(t * wr1b_ref[...], axis=1)
  gid = lax.broadcasted_iota(jnp.int32, (NBLK, H), 1)
  mask = batch_ref[...] == gid
  contrib = jnp.sum(jnp.where(mask, en[:, None], 0.0), axis=0)

  @pl.when(pl.program_id(0) == 0)
  def _():
    e_ref[...] = jnp.zeros_like(e_ref)

  e_ref[...] += contrib[None, :]


def _c1(pagg, nf1, batch2d, wout1, wsc1, wr1a, wr1brow):
  nb = lambda i: (i, 0)
  wfull = lambda i: (0, 0)
  return pl.pallas_call(
      _c1_body,
      grid=(N // NBLK,),
      in_specs=[
          pl.BlockSpec((NC, NBLK, H), lambda i: (0, i, 0)),
          pl.BlockSpec((NBLK, H), nb),
          pl.BlockSpec((NBLK, 1), nb),
          pl.BlockSpec((H, H), wfull),
          pl.BlockSpec((H, H), wfull),
          pl.BlockSpec((H, 16), wfull),
          pl.BlockSpec((1, 16), wfull),
      ],
      out_specs=pl.BlockSpec((1, H), wfull),
      out_shape=jax.ShapeDtypeStruct((1, H), jnp.float32),
  )(pagg, nf1, batch2d, wout1, wsc1, wr1a, wr1brow)


def kernel(positions, node_attrs, shifts, atomic_energies, W_embed, W_up,
           R_W1, R_W2, w_sh, W_out, W_sc, W_r0, W_r1a, W_r1b,
           edge_index, batch):
  f32 = jnp.float32
  snd = edge_index[0].astype(jnp.int32)
  rcv = edge_index[1].astype(jnp.int32)
  snd3 = snd.reshape(NW, NCHUNK, CHUNK)
  rcv3 = rcv.reshape(NW, NCHUNK, CHUNK)
  pad_c = ((0, 0), (0, SNCHUNK_PAD - SNCHUNK), (0, 0))
  snd3s = jnp.pad(snd.reshape(NW, SNCHUNK, SCHUNK), pad_c)
  rcv3s = jnp.pad(rcv.reshape(NW, SNCHUNK, SCHUNK), pad_c)

  pos_p = jnp.pad(positions.astype(f32), ((0, 0), (0, 13)))
  na_p = jnp.pad(node_attrs.astype(f32), ((0, 0), (0, 6)))
  batch2d = batch.astype(jnp.int32).reshape(N, 1)
  zeros_nh = jnp.zeros((NPAD, H), f32)

  wupe = jnp.pad(W_embed @ W_up[0], ((0, 6), (0, 0)))
  wsce = jnp.pad(W_embed @ W_sc[0], ((0, 6), (0, 0)))
  aerow = jnp.pad(atomic_energies.reshape(1, -1), ((0, 0), (0, 6)))
  wr0row = W_r0.reshape(1, H)
  wr1brow = W_r1b.reshape(1, 16)
  wv = w_sh[:, 1:4] * SQRT3
  w0 = w_sh[:, 0]

  d_lin = _scg(pos_p, snd3, rcv3)
  rp0, rp1 = _edge_tc(d_lin, shifts.astype(f32), R_W1[0], R_W2[0],
                      R_W1[1], R_W2[1], wv, w0)
  h0 = _h0(na_p, wupe)
  pagg0 = _scatter(h0, rp0, snd3s, rcv3s, zeros_nh)
  nf1, h1, e0 = _c0(pagg0, na_p, batch2d, W_out[0], wsce, wr0row, aerow,
                    W_up[1])
  pagg1 = _scatter(h1, rp1, snd3s, rcv3s, zeros_nh)
  e1 = _c1(pagg1, nf1, batch2d, W_out[1], W_sc[1], W_r1a, wr1brow)

  return (e0 + e1)[0, :NUM_GRAPHS]
